# baseline (speedup 1.0000x reference)
import functools

import jax
import jax.numpy as jnp
from jax import lax
from jax.experimental import pallas as pl
from jax.experimental.pallas import tpu as pltpu
from jax.experimental.pallas import tpu_sc as plsc

N = 10000
E = 320000
D = 128
H = 256
T = 2

NC = 2
NS = 16
NW = NC * NS
S = 2
ESEG = E // S
EPW = ESEG // NW
CHUNK = 40
NCHUNKS = EPW // CHUNK
N_PAD = 10240
RPS = N_PAD // NS

_mesh = plsc.VectorSubcoreMesh(core_axis_name="c", subcore_axis_name="s")


@functools.partial(
    pl.kernel,
    out_type=jax.ShapeDtypeStruct((ESEG, H), jnp.float32),
    mesh=_mesh,
    scratch_types=[
        pltpu.VMEM((NCHUNKS, CHUNK), jnp.int32),
        pltpu.VMEM((NCHUNKS, CHUNK), jnp.int32),
        pltpu.VMEM((CHUNK, H), jnp.float32),
        pltpu.VMEM((CHUNK, H), jnp.float32),
        pltpu.VMEM((CHUNK, H), jnp.float32),
        pltpu.VMEM((CHUNK, H), jnp.float32),
        pltpu.SemaphoreType.DMA,
        pltpu.SemaphoreType.DMA,
        pltpu.SemaphoreType.DMA,
        pltpu.SemaphoreType.DMA,
    ],
)
def _edge_gather(a_hbm, b_hbm, src3, dst3, z_hbm,
                 idxs, idxd, ar0, br0, ar1, br1, sg0, sg1, sw0, sw1):
    c = lax.axis_index("c")
    s = lax.axis_index("s")
    wid = s * NC + c
    base = wid * EPW
    pltpu.sync_copy(src3.at[wid], idxs)
    pltpu.sync_copy(dst3.at[wid], idxd)

    def start_gather(i, arows, brows, sem):
        pltpu.async_copy(a_hbm.at[idxs.at[i]], arows, sem)
        pltpu.async_copy(b_hbm.at[idxd.at[i]], brows, sem)

    def wait_gather(i, arows, brows, sem):
        pltpu.make_async_copy(a_hbm.at[idxs.at[i]], arows, sem).wait()
        pltpu.make_async_copy(b_hbm.at[idxd.at[i]], brows, sem).wait()

    def compute(arows, brows):
        def row_body(r, rc):
            for j in range(H // 16):
                va = arows[r, pl.ds(j * 16, 16)]
                vb = brows[r, pl.ds(j * 16, 16)]
                arows[r, pl.ds(j * 16, 16)] = jnp.maximum(va + vb, 0.0)
            return rc
        lax.fori_loop(0, CHUNK, row_body, 0, unroll=False)

    def zslice(i):
        return z_hbm.at[pl.ds(base + i * CHUNK, CHUNK)]

    start_gather(0, ar0, br0, sg0)

    @pl.loop(0, NCHUNKS - 1, step=2)
    def pair_body(iv):
        @pl.when(iv > 0)
        def _():
            pltpu.make_async_copy(ar1, zslice(iv - 1), sw1).wait()
        start_gather(iv + 1, ar1, br1, sg1)
        wait_gather(iv, ar0, br0, sg0)
        compute(ar0, br0)
        pltpu.async_copy(ar0, zslice(iv), sw0)
        pltpu.make_async_copy(ar0, zslice(iv), sw0).wait()
        start_gather(iv + 2, ar0, br0, sg0)
        wait_gather(iv + 1, ar1, br1, sg1)
        compute(ar1, br1)
        pltpu.async_copy(ar1, zslice(iv + 1), sw1)

    last = NCHUNKS - 1
    pltpu.make_async_copy(ar1, zslice(last - 1), sw1).wait()
    wait_gather(last, ar0, br0, sg0)
    compute(ar0, br0)
    pltpu.sync_copy(ar0, zslice(last))


@functools.partial(
    pl.kernel,
    out_type=jax.ShapeDtypeStruct((NC, N_PAD, D), jnp.float32),
    mesh=_mesh,
    scratch_types=[
        pltpu.VMEM((NCHUNKS, CHUNK), jnp.int32),
        pltpu.VMEM((CHUNK, D), jnp.float32),
        pltpu.VMEM((CHUNK, D), jnp.float32),
        pltpu.VMEM_SHARED((N_PAD, D), jnp.float32),
        pltpu.SemaphoreType.DMA,
        pltpu.SemaphoreType.DMA,
    ],
)
def _edge_scatter(m_hbm, dst3, zero_hbm, out_hbm, idxd, mr0, mr1, acc, sm0, sm1):
    c = lax.axis_index("c")
    s = lax.axis_index("s")
    wid = s * NC + c
    base = wid * EPW

    pltpu.sync_copy(zero_hbm.at[pl.ds(s * RPS, RPS)], acc.at[pl.ds(s * RPS, RPS)])
    pltpu.sync_copy(dst3.at[wid], idxd)
    plsc.subcore_barrier()

    def mslice(i):
        return m_hbm.at[pl.ds(base + i * CHUNK, CHUNK)]

    pltpu.async_copy(mslice(0), mr0, sm0)

    @pl.loop(0, NCHUNKS - 1, step=2)
    def pair_body(iv):
        pltpu.async_copy(mslice(iv + 1), mr1, sm1)
        pltpu.make_async_copy(mslice(iv), mr0, sm0).wait()
        pltpu.sync_copy(mr0, acc.at[idxd.at[iv]], add=True)
        pltpu.async_copy(mslice(iv + 2), mr0, sm0)
        pltpu.make_async_copy(mslice(iv + 1), mr1, sm1).wait()
        pltpu.sync_copy(mr1, acc.at[idxd.at[iv + 1]], add=True)

    last = NCHUNKS - 1
    pltpu.make_async_copy(mslice(last), mr0, sm0).wait()
    pltpu.sync_copy(mr0, acc.at[idxd.at[last]], add=True)

    plsc.subcore_barrier()
    pltpu.sync_copy(acc.at[pl.ds(s * RPS, RPS)], out_hbm.at[c, pl.ds(s * RPS, RPS)])


def _first_stage(x, w1a, w1b, b1r):
    br = 2000

    def body(x_ref, wa_ref, wb_ref, b_ref, a_ref, bo_ref):
        xv = x_ref[...]
        a_ref[...] = jnp.dot(xv, wa_ref[...], preferred_element_type=jnp.float32)
        bo_ref[...] = jnp.dot(xv, wb_ref[...], preferred_element_type=jnp.float32) + b_ref[...]

    return pl.pallas_call(
        body,
        grid=(N // br,),
        in_specs=[
            pl.BlockSpec((br, D), lambda i: (i, 0)),
            pl.BlockSpec((D, H), lambda i: (0, 0)),
            pl.BlockSpec((D, H), lambda i: (0, 0)),
            pl.BlockSpec((1, H), lambda i: (0, 0)),
        ],
        out_specs=[
            pl.BlockSpec((br, H), lambda i: (i, 0)),
            pl.BlockSpec((br, H), lambda i: (i, 0)),
        ],
        out_shape=[
            jax.ShapeDtypeStruct((N, H), jnp.float32),
            jax.ShapeDtypeStruct((N, H), jnp.float32),
        ],
    )(x, w1a, w1b, b1r)


def _message_mm(z, w2, b2r):
    br = 2000

    def body(z_ref, w_ref, b_ref, o_ref):
        o_ref[...] = jnp.dot(z_ref[...], w_ref[...], preferred_element_type=jnp.float32) + b_ref[...]

    return pl.pallas_call(
        body,
        grid=(ESEG // br,),
        in_specs=[
            pl.BlockSpec((br, H), lambda i: (i, 0)),
            pl.BlockSpec((H, D), lambda i: (0, 0)),
            pl.BlockSpec((1, D), lambda i: (0, 0)),
        ],
        out_specs=pl.BlockSpec((br, D), lambda i: (i, 0)),
        out_shape=jax.ShapeDtypeStruct((ESEG, D), jnp.float32),
    )(z, w2, b2r)


def _update_next(p0, p1, p2, p3, h, wu1a, wu1b, bu1r, wu2, bu2r, w1a, w1b, b1r):
    br = 2000

    def body(p0_ref, p1_ref, p2_ref, p3_ref, h_ref, wu1a_ref, wu1b_ref,
             bu1_ref, wu2_ref, bu2_ref, w1a_ref, w1b_ref, b1_ref,
             h_out, a_out, b_out):
        agg = (p0_ref[...] + p1_ref[...]) + (p2_ref[...] + p3_ref[...])
        u = jnp.maximum(
            jnp.dot(agg, wu1a_ref[...], preferred_element_type=jnp.float32)
            + jnp.dot(h_ref[...], wu1b_ref[...], preferred_element_type=jnp.float32)
            + bu1_ref[...], 0.0)
        hn = jnp.dot(u, wu2_ref[...], preferred_element_type=jnp.float32) + bu2_ref[...]
        h_out[...] = hn
        a_out[...] = jnp.dot(hn, w1a_ref[...], preferred_element_type=jnp.float32)
        b_out[...] = jnp.dot(hn, w1b_ref[...], preferred_element_type=jnp.float32) + b1_ref[...]

    full = lambda i: (0, 0)
    return pl.pallas_call(
        body,
        grid=(N // br,),
        in_specs=[
            pl.BlockSpec((br, D), lambda i: (i, 0)),
            pl.BlockSpec((br, D), lambda i: (i, 0)),
            pl.BlockSpec((br, D), lambda i: (i, 0)),
            pl.BlockSpec((br, D), lambda i: (i, 0)),
            pl.BlockSpec((br, D), lambda i: (i, 0)),
            pl.BlockSpec((D, H), full),
            pl.BlockSpec((D, H), full),
            pl.BlockSpec((1, H), full),
            pl.BlockSpec((H, D), full),
            pl.BlockSpec((1, D), full),
            pl.BlockSpec((D, H), full),
            pl.BlockSpec((D, H), full),
            pl.BlockSpec((1, H), full),
        ],
        out_specs=[
            pl.BlockSpec((br, D), lambda i: (i, 0)),
            pl.BlockSpec((br, H), lambda i: (i, 0)),
            pl.BlockSpec((br, H), lambda i: (i, 0)),
        ],
        out_shape=[
            jax.ShapeDtypeStruct((N, D), jnp.float32),
            jax.ShapeDtypeStruct((N, H), jnp.float32),
            jax.ShapeDtypeStruct((N, H), jnp.float32),
        ],
    )(p0, p1, p2, p3, h, wu1a, wu1b, bu1r, wu2, bu2r, w1a, w1b, b1r)


def _update_readout(p0, p1, p2, p3, h, wu1a, wu1b, bu1r, wu2, bu2r, wr_pad, br_pad):
    br = 2000

    def body(p0_ref, p1_ref, p2_ref, p3_ref, h_ref, wu1a_ref, wu1b_ref,
             bu1_ref, wu2_ref, bu2_ref, wr_ref, brd_ref, o_ref):
        agg = (p0_ref[...] + p1_ref[...]) + (p2_ref[...] + p3_ref[...])
        u = jnp.maximum(
            jnp.dot(agg, wu1a_ref[...], preferred_element_type=jnp.float32)
            + jnp.dot(h_ref[...], wu1b_ref[...], preferred_element_type=jnp.float32)
            + bu1_ref[...], 0.0)
        hn = jnp.dot(u, wu2_ref[...], preferred_element_type=jnp.float32) + bu2_ref[...]
        o_ref[...] = jnp.dot(hn, wr_ref[...], preferred_element_type=jnp.float32) + brd_ref[...]

    full = lambda i: (0, 0)
    return pl.pallas_call(
        body,
        grid=(N // br,),
        in_specs=[
            pl.BlockSpec((br, D), lambda i: (i, 0)),
            pl.BlockSpec((br, D), lambda i: (i, 0)),
            pl.BlockSpec((br, D), lambda i: (i, 0)),
            pl.BlockSpec((br, D), lambda i: (i, 0)),
            pl.BlockSpec((br, D), lambda i: (i, 0)),
            pl.BlockSpec((D, H), full),
            pl.BlockSpec((D, H), full),
            pl.BlockSpec((1, H), full),
            pl.BlockSpec((H, D), full),
            pl.BlockSpec((1, D), full),
            pl.BlockSpec((D, D), full),
            pl.BlockSpec((1, D), full),
        ],
        out_specs=pl.BlockSpec((br, D), lambda i: (i, 0)),
        out_shape=jax.ShapeDtypeStruct((N, D), jnp.float32),
    )(p0, p1, p2, p3, h, wu1a, wu1b, bu1r, wu2, bu2r, wr_pad, br_pad)


def kernel(x, edge_index, W1, b1, W2, b2, Wu1, bu1, Wu2, bu2, Wr, br):
    src4 = edge_index[0].reshape(S, NW, NCHUNKS, CHUNK)
    dst4 = edge_index[1].reshape(S, NW, NCHUNKS, CHUNK)
    w1a, w1b = W1[:D], W1[D:]
    wu1a, wu1b = Wu1[:D], Wu1[D:]
    b1r = b1.reshape(1, H)
    b2r = b2.reshape(1, D)
    bu1r = bu1.reshape(1, H)
    bu2r = bu2.reshape(1, D)
    wr_pad = jnp.pad(Wr, ((0, 0), (0, D - 1)))
    br_pad = jnp.pad(br, (0, D - 1)).reshape(1, D)
    zero_nd = jnp.zeros((N_PAD, D), jnp.float32)

    h = x
    a, b = _first_stage(x, w1a, w1b, b1r)
    for t in range(T):
        ps = []
        for sidx in range(S):
            z = _edge_gather(a, b, src4[sidx], dst4[sidx])
            m = _message_mm(z, W2, b2r)
            parts = _edge_scatter(m, dst4[sidx], zero_nd)
            ps += [parts[0, :N], parts[1, :N]]
        if t < T - 1:
            h, a, b = _update_next(ps[0], ps[1], ps[2], ps[3], h,
                                   wu1a, wu1b, bu1r, Wu2, bu2r, w1a, w1b, b1r)
        else:
            out_full = _update_readout(ps[0], ps[1], ps[2], ps[3], h,
                                       wu1a, wu1b, bu1r, Wu2, bu2r, wr_pad, br_pad)
    return out_full[:, :1]


# speedup vs baseline: 1.0146x; 1.0146x over previous
import functools

import jax
import jax.numpy as jnp
from jax import lax
from jax.experimental import pallas as pl
from jax.experimental.pallas import tpu as pltpu
from jax.experimental.pallas import tpu_sc as plsc

N = 10000
E = 320000
D = 128
H = 256
T = 2

NC = 2
NS = 16
NW = NC * NS
S = 2
ESEG = E // S
EPW = ESEG // NW
CHUNK = 40
NCHUNKS = EPW // CHUNK
N_PAD = 10240
RPS = N_PAD // NS

_mesh = plsc.VectorSubcoreMesh(core_axis_name="c", subcore_axis_name="s")


@functools.partial(
    pl.kernel,
    out_type=jax.ShapeDtypeStruct((ESEG, H), jnp.float32),
    mesh=_mesh,
    scratch_types=[
        pltpu.VMEM((NCHUNKS, CHUNK), jnp.int32),
        pltpu.VMEM((NCHUNKS, CHUNK), jnp.int32),
        pltpu.VMEM((CHUNK, H), jnp.float32),
        pltpu.VMEM((CHUNK, H), jnp.float32),
        pltpu.VMEM((CHUNK, H), jnp.float32),
        pltpu.VMEM((CHUNK, H), jnp.float32),
        pltpu.SemaphoreType.DMA,
        pltpu.SemaphoreType.DMA,
        pltpu.SemaphoreType.DMA,
        pltpu.SemaphoreType.DMA,
    ],
)
def _edge_gather(a_hbm, b_hbm, src3, dst3, z_hbm,
                 idxs, idxd, ar0, br0, ar1, br1, sg0, sg1, sw0, sw1):
    c = lax.axis_index("c")
    s = lax.axis_index("s")
    wid = s * NC + c
    base = wid * EPW
    pltpu.sync_copy(src3.at[wid], idxs)
    pltpu.sync_copy(dst3.at[wid], idxd)

    def start_gather(i, arows, brows, sem):
        pltpu.async_copy(a_hbm.at[idxs.at[i]], arows, sem)
        pltpu.async_copy(b_hbm.at[idxd.at[i]], brows, sem)

    def wait_gather(i, arows, brows, sem):
        pltpu.make_async_copy(a_hbm.at[idxs.at[i]], arows, sem).wait()
        pltpu.make_async_copy(b_hbm.at[idxd.at[i]], brows, sem).wait()

    def compute(arows, brows):
        def row_body(r, rc):
            for j in range(H // 16):
                va = arows[r, pl.ds(j * 16, 16)]
                vb = brows[r, pl.ds(j * 16, 16)]
                arows[r, pl.ds(j * 16, 16)] = jnp.maximum(va + vb, 0.0)
            return rc
        lax.fori_loop(0, CHUNK, row_body, 0, unroll=False)

    def zslice(i):
        return z_hbm.at[pl.ds(base + i * CHUNK, CHUNK)]

    start_gather(0, ar0, br0, sg0)

    @pl.loop(0, NCHUNKS - 1, step=2)
    def pair_body(iv):
        @pl.when(iv > 0)
        def _():
            pltpu.make_async_copy(ar1, zslice(iv - 1), sw1).wait()
        start_gather(iv + 1, ar1, br1, sg1)
        wait_gather(iv, ar0, br0, sg0)
        compute(ar0, br0)
        pltpu.async_copy(ar0, zslice(iv), sw0)
        pltpu.make_async_copy(ar0, zslice(iv), sw0).wait()
        start_gather(iv + 2, ar0, br0, sg0)
        wait_gather(iv + 1, ar1, br1, sg1)
        compute(ar1, br1)
        pltpu.async_copy(ar1, zslice(iv + 1), sw1)

    last = NCHUNKS - 1
    pltpu.make_async_copy(ar1, zslice(last - 1), sw1).wait()
    wait_gather(last, ar0, br0, sg0)
    compute(ar0, br0)
    pltpu.sync_copy(ar0, zslice(last))


@functools.partial(
    pl.kernel,
    out_type=jax.ShapeDtypeStruct((NC, N_PAD, D), jnp.float32),
    mesh=_mesh,
    scratch_types=[
        pltpu.VMEM((NCHUNKS, CHUNK), jnp.int32),
        pltpu.VMEM((CHUNK, D), jnp.float32),
        pltpu.VMEM((CHUNK, D), jnp.float32),
        pltpu.VMEM_SHARED((N_PAD, D), jnp.float32),
        pltpu.SemaphoreType.DMA,
        pltpu.SemaphoreType.DMA,
    ],
)
def _edge_scatter(m_hbm, dst3, init_hbm, out_hbm, idxd, mr0, mr1, acc, sm0, sm1):
    c = lax.axis_index("c")
    s = lax.axis_index("s")
    wid = s * NC + c
    base = wid * EPW

    pltpu.sync_copy(init_hbm.at[c, pl.ds(s * RPS, RPS)], acc.at[pl.ds(s * RPS, RPS)])
    pltpu.sync_copy(dst3.at[wid], idxd)
    plsc.subcore_barrier()

    def mslice(i):
        return m_hbm.at[pl.ds(base + i * CHUNK, CHUNK)]

    pltpu.async_copy(mslice(0), mr0, sm0)

    @pl.loop(0, NCHUNKS - 1, step=2)
    def pair_body(iv):
        pltpu.async_copy(mslice(iv + 1), mr1, sm1)
        pltpu.make_async_copy(mslice(iv), mr0, sm0).wait()
        pltpu.sync_copy(mr0, acc.at[idxd.at[iv]], add=True)
        pltpu.async_copy(mslice(iv + 2), mr0, sm0)
        pltpu.make_async_copy(mslice(iv + 1), mr1, sm1).wait()
        pltpu.sync_copy(mr1, acc.at[idxd.at[iv + 1]], add=True)

    last = NCHUNKS - 1
    pltpu.make_async_copy(mslice(last), mr0, sm0).wait()
    pltpu.sync_copy(mr0, acc.at[idxd.at[last]], add=True)

    plsc.subcore_barrier()
    pltpu.sync_copy(acc.at[pl.ds(s * RPS, RPS)], out_hbm.at[c, pl.ds(s * RPS, RPS)])


def _first_stage(x, w1a, w1b, b1r):
    br = 2000

    def body(x_ref, wa_ref, wb_ref, b_ref, a_ref, bo_ref):
        xv = x_ref[...]
        a_ref[...] = jnp.dot(xv, wa_ref[...], preferred_element_type=jnp.float32)
        bo_ref[...] = jnp.dot(xv, wb_ref[...], preferred_element_type=jnp.float32) + b_ref[...]

    return pl.pallas_call(
        body,
        grid=(N // br,),
        in_specs=[
            pl.BlockSpec((br, D), lambda i: (i, 0)),
            pl.BlockSpec((D, H), lambda i: (0, 0)),
            pl.BlockSpec((D, H), lambda i: (0, 0)),
            pl.BlockSpec((1, H), lambda i: (0, 0)),
        ],
        out_specs=[
            pl.BlockSpec((br, H), lambda i: (i, 0)),
            pl.BlockSpec((br, H), lambda i: (i, 0)),
        ],
        out_shape=[
            jax.ShapeDtypeStruct((N, H), jnp.float32),
            jax.ShapeDtypeStruct((N, H), jnp.float32),
        ],
    )(x, w1a, w1b, b1r)


def _message_mm(z, w2, b2r):
    br = 2000

    def body(z_ref, w_ref, b_ref, o_ref):
        o_ref[...] = jnp.dot(z_ref[...], w_ref[...], preferred_element_type=jnp.float32) + b_ref[...]

    return pl.pallas_call(
        body,
        grid=(ESEG // br,),
        in_specs=[
            pl.BlockSpec((br, H), lambda i: (i, 0)),
            pl.BlockSpec((H, D), lambda i: (0, 0)),
            pl.BlockSpec((1, D), lambda i: (0, 0)),
        ],
        out_specs=pl.BlockSpec((br, D), lambda i: (i, 0)),
        out_shape=jax.ShapeDtypeStruct((ESEG, D), jnp.float32),
    )(z, w2, b2r)


def _update_next(p0, p1, h, wu1a, wu1b, bu1r, wu2, bu2r, w1a, w1b, b1r):
    br = 2000

    def body(p0_ref, p1_ref, h_ref, wu1a_ref, wu1b_ref,
             bu1_ref, wu2_ref, bu2_ref, w1a_ref, w1b_ref, b1_ref,
             h_out, a_out, b_out):
        agg = p0_ref[...] + p1_ref[...]
        u = jnp.maximum(
            jnp.dot(agg, wu1a_ref[...], preferred_element_type=jnp.float32)
            + jnp.dot(h_ref[...], wu1b_ref[...], preferred_element_type=jnp.float32)
            + bu1_ref[...], 0.0)
        hn = jnp.dot(u, wu2_ref[...], preferred_element_type=jnp.float32) + bu2_ref[...]
        h_out[...] = hn
        a_out[...] = jnp.dot(hn, w1a_ref[...], preferred_element_type=jnp.float32)
        b_out[...] = jnp.dot(hn, w1b_ref[...], preferred_element_type=jnp.float32) + b1_ref[...]

    full = lambda i: (0, 0)
    return pl.pallas_call(
        body,
        grid=(N // br,),
        in_specs=[
            pl.BlockSpec((br, D), lambda i: (i, 0)),
            pl.BlockSpec((br, D), lambda i: (i, 0)),
            pl.BlockSpec((br, D), lambda i: (i, 0)),
            pl.BlockSpec((D, H), full),
            pl.BlockSpec((D, H), full),
            pl.BlockSpec((1, H), full),
            pl.BlockSpec((H, D), full),
            pl.BlockSpec((1, D), full),
            pl.BlockSpec((D, H), full),
            pl.BlockSpec((D, H), full),
            pl.BlockSpec((1, H), full),
        ],
        out_specs=[
            pl.BlockSpec((br, D), lambda i: (i, 0)),
            pl.BlockSpec((br, H), lambda i: (i, 0)),
            pl.BlockSpec((br, H), lambda i: (i, 0)),
        ],
        out_shape=[
            jax.ShapeDtypeStruct((N, D), jnp.float32),
            jax.ShapeDtypeStruct((N, H), jnp.float32),
            jax.ShapeDtypeStruct((N, H), jnp.float32),
        ],
    )(p0, p1, h, wu1a, wu1b, bu1r, wu2, bu2r, w1a, w1b, b1r)


def _update_readout(p0, p1, h, wu1a, wu1b, bu1r, wu2, bu2r, wr_pad, br_pad):
    br = 2000

    def body(p0_ref, p1_ref, h_ref, wu1a_ref, wu1b_ref,
             bu1_ref, wu2_ref, bu2_ref, wr_ref, brd_ref, o_ref):
        agg = p0_ref[...] + p1_ref[...]
        u = jnp.maximum(
            jnp.dot(agg, wu1a_ref[...], preferred_element_type=jnp.float32)
            + jnp.dot(h_ref[...], wu1b_ref[...], preferred_element_type=jnp.float32)
            + bu1_ref[...], 0.0)
        hn = jnp.dot(u, wu2_ref[...], preferred_element_type=jnp.float32) + bu2_ref[...]
        o_ref[...] = jnp.dot(hn, wr_ref[...], preferred_element_type=jnp.float32) + brd_ref[...]

    full = lambda i: (0, 0)
    return pl.pallas_call(
        body,
        grid=(N // br,),
        in_specs=[
            pl.BlockSpec((br, D), lambda i: (i, 0)),
            pl.BlockSpec((br, D), lambda i: (i, 0)),
            pl.BlockSpec((br, D), lambda i: (i, 0)),
            pl.BlockSpec((D, H), full),
            pl.BlockSpec((D, H), full),
            pl.BlockSpec((1, H), full),
            pl.BlockSpec((H, D), full),
            pl.BlockSpec((1, D), full),
            pl.BlockSpec((D, D), full),
            pl.BlockSpec((1, D), full),
        ],
        out_specs=pl.BlockSpec((br, D), lambda i: (i, 0)),
        out_shape=jax.ShapeDtypeStruct((N, D), jnp.float32),
    )(p0, p1, h, wu1a, wu1b, bu1r, wu2, bu2r, wr_pad, br_pad)


def kernel(x, edge_index, W1, b1, W2, b2, Wu1, bu1, Wu2, bu2, Wr, br):
    src4 = edge_index[0].reshape(S, NW, NCHUNKS, CHUNK)
    dst4 = edge_index[1].reshape(S, NW, NCHUNKS, CHUNK)
    w1a, w1b = W1[:D], W1[D:]
    wu1a, wu1b = Wu1[:D], Wu1[D:]
    b1r = b1.reshape(1, H)
    b2r = b2.reshape(1, D)
    bu1r = bu1.reshape(1, H)
    bu2r = bu2.reshape(1, D)
    wr_pad = jnp.pad(Wr, ((0, 0), (0, D - 1)))
    br_pad = jnp.pad(br, (0, D - 1)).reshape(1, D)
    zero_nd = jnp.zeros((NC, N_PAD, D), jnp.float32)

    h = x
    a, b = _first_stage(x, w1a, w1b, b1r)
    for t in range(T):
        parts = zero_nd
        for sidx in range(S):
            z = _edge_gather(a, b, src4[sidx], dst4[sidx])
            m = _message_mm(z, W2, b2r)
            parts = _edge_scatter(m, dst4[sidx], parts)
        p0, p1 = parts[0, :N], parts[1, :N]
        if t < T - 1:
            h, a, b = _update_next(p0, p1, h, wu1a, wu1b, bu1r, Wu2, bu2r,
                                   w1a, w1b, b1r)
        else:
            out_full = _update_readout(p0, p1, h, wu1a, wu1b, bu1r, Wu2, bu2r,
                                       wr_pad, br_pad)
    return out_full[:, :1]


# speedup vs baseline: 1.0587x; 1.0434x over previous
import functools

import jax
import jax.numpy as jnp
from jax import lax
from jax.experimental import pallas as pl
from jax.experimental.pallas import tpu as pltpu
from jax.experimental.pallas import tpu_sc as plsc

N = 10000
E = 320000
D = 128
H = 256
T = 2

NC = 2
NS = 16
NW = NC * NS
S = 2
ESEG = E // S
EPW = ESEG // NW
CHUNK = 40
NCHUNKS = EPW // CHUNK
N_PAD = 10240
RPS = N_PAD // NS

_mesh = plsc.VectorSubcoreMesh(core_axis_name="c", subcore_axis_name="s")


@functools.partial(
    pl.kernel,
    out_type=jax.ShapeDtypeStruct((ESEG, H), jnp.float32),
    mesh=_mesh,
    scratch_types=[
        pltpu.VMEM((NCHUNKS, CHUNK), jnp.int32),
        pltpu.VMEM((NCHUNKS, CHUNK), jnp.int32),
        [pltpu.VMEM((CHUNK, H), jnp.float32)] * 4,
        [pltpu.VMEM((CHUNK, H), jnp.float32)] * 4,
        [pltpu.SemaphoreType.DMA] * 4,
        [pltpu.SemaphoreType.DMA] * 4,
    ],
)
def _edge_gather(a_hbm, b_hbm, src3, dst3, z_hbm, idxs, idxd, ar, br_, sg, sw):
    c = lax.axis_index("c")
    s = lax.axis_index("s")
    wid = s * NC + c
    base = wid * EPW
    pltpu.sync_copy(src3.at[wid], idxs)
    pltpu.sync_copy(dst3.at[wid], idxd)

    def start_gather(i, p):
        pltpu.async_copy(a_hbm.at[idxs.at[i]], ar[p], sg[p])
        pltpu.async_copy(b_hbm.at[idxd.at[i]], br_[p], sg[p])

    def wait_gather(i, p):
        pltpu.make_async_copy(a_hbm.at[idxs.at[i]], ar[p], sg[p]).wait()
        pltpu.make_async_copy(b_hbm.at[idxd.at[i]], br_[p], sg[p]).wait()

    def compute(p):
        arows, brows = ar[p], br_[p]

        def row_body(r, rc):
            for j in range(H // 16):
                va = arows[r, pl.ds(j * 16, 16)]
                vb = brows[r, pl.ds(j * 16, 16)]
                arows[r, pl.ds(j * 16, 16)] = jnp.maximum(va + vb, 0.0)
            return rc
        lax.fori_loop(0, CHUNK, row_body, 0, unroll=False)

    def zslice(i):
        return z_hbm.at[pl.ds(base + i * CHUNK, CHUNK)]

    for p in range(3):
        start_gather(p, p)

    @pl.loop(0, NCHUNKS - 1, step=4)
    def ring_body(iv):
        for k in range(4):
            p = k % 4
            q = (k + 3) % 4
            j = iv + k
            wait_gather(j, p)
            compute(p)
            pltpu.async_copy(ar[p], zslice(j), sw[p])
            if k == 0:
                @pl.when(iv > 0)
                def _():
                    pltpu.make_async_copy(ar[q], zslice(j - 1), sw[q]).wait()
            else:
                pltpu.make_async_copy(ar[q], zslice(j - 1), sw[q]).wait()

            @pl.when(j < NCHUNKS - 3)
            def _():
                start_gather(j + 3, q)

    last = NCHUNKS - 1
    pltpu.make_async_copy(ar[3], zslice(last - 1), sw[3]).wait()
    wait_gather(last, 0)
    compute(0)
    pltpu.sync_copy(ar[0], zslice(last))


@functools.partial(
    pl.kernel,
    out_type=jax.ShapeDtypeStruct((NC, N_PAD, D), jnp.float32),
    mesh=_mesh,
    scratch_types=[
        pltpu.VMEM((NCHUNKS, CHUNK), jnp.int32),
        pltpu.VMEM((CHUNK, D), jnp.float32),
        pltpu.VMEM((CHUNK, D), jnp.float32),
        pltpu.VMEM_SHARED((N_PAD, D), jnp.float32),
        pltpu.SemaphoreType.DMA,
        pltpu.SemaphoreType.DMA,
    ],
)
def _edge_scatter(m_hbm, dst3, init_hbm, out_hbm, idxd, mr0, mr1, acc, sm0, sm1):
    c = lax.axis_index("c")
    s = lax.axis_index("s")
    wid = s * NC + c
    base = wid * EPW

    pltpu.sync_copy(init_hbm.at[c, pl.ds(s * RPS, RPS)], acc.at[pl.ds(s * RPS, RPS)])
    pltpu.sync_copy(dst3.at[wid], idxd)
    plsc.subcore_barrier()

    def mslice(i):
        return m_hbm.at[pl.ds(base + i * CHUNK, CHUNK)]

    pltpu.async_copy(mslice(0), mr0, sm0)

    @pl.loop(0, NCHUNKS - 1, step=2)
    def pair_body(iv):
        pltpu.async_copy(mslice(iv + 1), mr1, sm1)
        pltpu.make_async_copy(mslice(iv), mr0, sm0).wait()
        pltpu.sync_copy(mr0, acc.at[idxd.at[iv]], add=True)
        pltpu.async_copy(mslice(iv + 2), mr0, sm0)
        pltpu.make_async_copy(mslice(iv + 1), mr1, sm1).wait()
        pltpu.sync_copy(mr1, acc.at[idxd.at[iv + 1]], add=True)

    last = NCHUNKS - 1
    pltpu.make_async_copy(mslice(last), mr0, sm0).wait()
    pltpu.sync_copy(mr0, acc.at[idxd.at[last]], add=True)

    plsc.subcore_barrier()
    pltpu.sync_copy(acc.at[pl.ds(s * RPS, RPS)], out_hbm.at[c, pl.ds(s * RPS, RPS)])


def _first_stage(x, w1a, w1b, b1r):
    br = 2000

    def body(x_ref, wa_ref, wb_ref, b_ref, a_ref, bo_ref):
        xv = x_ref[...]
        a_ref[...] = jnp.dot(xv, wa_ref[...], preferred_element_type=jnp.float32)
        bo_ref[...] = jnp.dot(xv, wb_ref[...], preferred_element_type=jnp.float32) + b_ref[...]

    return pl.pallas_call(
        body,
        grid=(N // br,),
        in_specs=[
            pl.BlockSpec((br, D), lambda i: (i, 0)),
            pl.BlockSpec((D, H), lambda i: (0, 0)),
            pl.BlockSpec((D, H), lambda i: (0, 0)),
            pl.BlockSpec((1, H), lambda i: (0, 0)),
        ],
        out_specs=[
            pl.BlockSpec((br, H), lambda i: (i, 0)),
            pl.BlockSpec((br, H), lambda i: (i, 0)),
        ],
        out_shape=[
            jax.ShapeDtypeStruct((N, H), jnp.float32),
            jax.ShapeDtypeStruct((N, H), jnp.float32),
        ],
    )(x, w1a, w1b, b1r)


def _message_mm(z, w2, b2r):
    br = 2000

    def body(z_ref, w_ref, b_ref, o_ref):
        o_ref[...] = jnp.dot(z_ref[...], w_ref[...], preferred_element_type=jnp.float32) + b_ref[...]

    return pl.pallas_call(
        body,
        grid=(ESEG // br,),
        in_specs=[
            pl.BlockSpec((br, H), lambda i: (i, 0)),
            pl.BlockSpec((H, D), lambda i: (0, 0)),
            pl.BlockSpec((1, D), lambda i: (0, 0)),
        ],
        out_specs=pl.BlockSpec((br, D), lambda i: (i, 0)),
        out_shape=jax.ShapeDtypeStruct((ESEG, D), jnp.float32),
    )(z, w2, b2r)


def _update_next(p0, p1, h, wu1a, wu1b, bu1r, wu2, bu2r, w1a, w1b, b1r):
    br = 2000

    def body(p0_ref, p1_ref, h_ref, wu1a_ref, wu1b_ref,
             bu1_ref, wu2_ref, bu2_ref, w1a_ref, w1b_ref, b1_ref,
             h_out, a_out, b_out):
        agg = p0_ref[...] + p1_ref[...]
        u = jnp.maximum(
            jnp.dot(agg, wu1a_ref[...], preferred_element_type=jnp.float32)
            + jnp.dot(h_ref[...], wu1b_ref[...], preferred_element_type=jnp.float32)
            + bu1_ref[...], 0.0)
        hn = jnp.dot(u, wu2_ref[...], preferred_element_type=jnp.float32) + bu2_ref[...]
        h_out[...] = hn
        a_out[...] = jnp.dot(hn, w1a_ref[...], preferred_element_type=jnp.float32)
        b_out[...] = jnp.dot(hn, w1b_ref[...], preferred_element_type=jnp.float32) + b1_ref[...]

    full = lambda i: (0, 0)
    return pl.pallas_call(
        body,
        grid=(N // br,),
        in_specs=[
            pl.BlockSpec((br, D), lambda i: (i, 0)),
            pl.BlockSpec((br, D), lambda i: (i, 0)),
            pl.BlockSpec((br, D), lambda i: (i, 0)),
            pl.BlockSpec((D, H), full),
            pl.BlockSpec((D, H), full),
            pl.BlockSpec((1, H), full),
            pl.BlockSpec((H, D), full),
            pl.BlockSpec((1, D), full),
            pl.BlockSpec((D, H), full),
            pl.BlockSpec((D, H), full),
            pl.BlockSpec((1, H), full),
        ],
        out_specs=[
            pl.BlockSpec((br, D), lambda i: (i, 0)),
            pl.BlockSpec((br, H), lambda i: (i, 0)),
            pl.BlockSpec((br, H), lambda i: (i, 0)),
        ],
        out_shape=[
            jax.ShapeDtypeStruct((N, D), jnp.float32),
            jax.ShapeDtypeStruct((N, H), jnp.float32),
            jax.ShapeDtypeStruct((N, H), jnp.float32),
        ],
    )(p0, p1, h, wu1a, wu1b, bu1r, wu2, bu2r, w1a, w1b, b1r)


def _update_readout(p0, p1, h, wu1a, wu1b, bu1r, wu2, bu2r, wr_pad, br_pad):
    br = 2000

    def body(p0_ref, p1_ref, h_ref, wu1a_ref, wu1b_ref,
             bu1_ref, wu2_ref, bu2_ref, wr_ref, brd_ref, o_ref):
        agg = p0_ref[...] + p1_ref[...]
        u = jnp.maximum(
            jnp.dot(agg, wu1a_ref[...], preferred_element_type=jnp.float32)
            + jnp.dot(h_ref[...], wu1b_ref[...], preferred_element_type=jnp.float32)
            + bu1_ref[...], 0.0)
        hn = jnp.dot(u, wu2_ref[...], preferred_element_type=jnp.float32) + bu2_ref[...]
        o_ref[...] = jnp.dot(hn, wr_ref[...], preferred_element_type=jnp.float32) + brd_ref[...]

    full = lambda i: (0, 0)
    return pl.pallas_call(
        body,
        grid=(N // br,),
        in_specs=[
            pl.BlockSpec((br, D), lambda i: (i, 0)),
            pl.BlockSpec((br, D), lambda i: (i, 0)),
            pl.BlockSpec((br, D), lambda i: (i, 0)),
            pl.BlockSpec((D, H), full),
            pl.BlockSpec((D, H), full),
            pl.BlockSpec((1, H), full),
            pl.BlockSpec((H, D), full),
            pl.BlockSpec((1, D), full),
            pl.BlockSpec((D, D), full),
            pl.BlockSpec((1, D), full),
        ],
        out_specs=pl.BlockSpec((br, D), lambda i: (i, 0)),
        out_shape=jax.ShapeDtypeStruct((N, D), jnp.float32),
    )(p0, p1, h, wu1a, wu1b, bu1r, wu2, bu2r, wr_pad, br_pad)


def kernel(x, edge_index, W1, b1, W2, b2, Wu1, bu1, Wu2, bu2, Wr, br):
    src4 = edge_index[0].reshape(S, NW, NCHUNKS, CHUNK)
    dst4 = edge_index[1].reshape(S, NW, NCHUNKS, CHUNK)
    w1a, w1b = W1[:D], W1[D:]
    wu1a, wu1b = Wu1[:D], Wu1[D:]
    b1r = b1.reshape(1, H)
    b2r = b2.reshape(1, D)
    bu1r = bu1.reshape(1, H)
    bu2r = bu2.reshape(1, D)
    wr_pad = jnp.pad(Wr, ((0, 0), (0, D - 1)))
    br_pad = jnp.pad(br, (0, D - 1)).reshape(1, D)
    zero_nd = jnp.zeros((NC, N_PAD, D), jnp.float32)

    h = x
    a, b = _first_stage(x, w1a, w1b, b1r)
    for t in range(T):
        parts = zero_nd
        for sidx in range(S):
            z = _edge_gather(a, b, src4[sidx], dst4[sidx])
            m = _message_mm(z, W2, b2r)
            parts = _edge_scatter(m, dst4[sidx], parts)
        p0, p1 = parts[0, :N], parts[1, :N]
        if t < T - 1:
            h, a, b = _update_next(p0, p1, h, wu1a, wu1b, bu1r, Wu2, bu2r,
                                   w1a, w1b, b1r)
        else:
            out_full = _update_readout(p0, p1, h, wu1a, wu1b, bu1r, Wu2, bu2r,
                                       wr_pad, br_pad)
    return out_full[:, :1]


# speedup vs baseline: 1.0896x; 1.0292x over previous
import functools

import jax
import jax.numpy as jnp
from jax import lax
from jax.experimental import pallas as pl
from jax.experimental.pallas import tpu as pltpu
from jax.experimental.pallas import tpu_sc as plsc

N = 10000
E = 320000
D = 128
H = 256
T = 2

NC = 2
NS = 16
NW = NC * NS
S = 2
ESEG = E // S
EPW = ESEG // NW
CHUNK = 40
NCHUNKS = EPW // CHUNK
SCHUNK = 40
SNCHUNKS = EPW // SCHUNK
N_PAD = 10240
RPS = N_PAD // NS

_mesh = plsc.VectorSubcoreMesh(core_axis_name="c", subcore_axis_name="s")


@functools.partial(
    pl.kernel,
    out_type=jax.ShapeDtypeStruct((ESEG, H), jnp.float32),
    mesh=_mesh,
    scratch_types=[
        pltpu.VMEM((NCHUNKS, CHUNK), jnp.int32),
        pltpu.VMEM((NCHUNKS, CHUNK), jnp.int32),
        [pltpu.VMEM((CHUNK, H), jnp.float32)] * 4,
        [pltpu.VMEM((CHUNK, H), jnp.float32)] * 4,
        [pltpu.SemaphoreType.DMA] * 4,
        [pltpu.SemaphoreType.DMA] * 4,
    ],
)
def _edge_gather(a_hbm, b_hbm, src3, dst3, z_hbm, idxs, idxd, ar, br_, sg, sw):
    c = lax.axis_index("c")
    s = lax.axis_index("s")
    wid = s * NC + c
    base = wid * EPW
    pltpu.sync_copy(src3.at[wid], idxs)
    pltpu.sync_copy(dst3.at[wid], idxd)

    def start_gather(i, p):
        pltpu.async_copy(a_hbm.at[idxs.at[i]], ar[p], sg[p])
        pltpu.async_copy(b_hbm.at[idxd.at[i]], br_[p], sg[p])

    def wait_gather(i, p):
        pltpu.make_async_copy(a_hbm.at[idxs.at[i]], ar[p], sg[p]).wait()
        pltpu.make_async_copy(b_hbm.at[idxd.at[i]], br_[p], sg[p]).wait()

    def compute(p):
        arows, brows = ar[p], br_[p]

        def row_body(r, rc):
            for j in range(H // 16):
                va = arows[r, pl.ds(j * 16, 16)]
                vb = brows[r, pl.ds(j * 16, 16)]
                arows[r, pl.ds(j * 16, 16)] = jnp.maximum(va + vb, 0.0)
            return rc
        lax.fori_loop(0, CHUNK, row_body, 0, unroll=False)

    def zslice(i):
        return z_hbm.at[pl.ds(base + i * CHUNK, CHUNK)]

    for p in range(3):
        start_gather(p, p)

    @pl.loop(0, NCHUNKS - 1, step=4)
    def ring_body(iv):
        for k in range(4):
            p = k % 4
            q = (k + 3) % 4
            j = iv + k
            wait_gather(j, p)
            compute(p)
            pltpu.async_copy(ar[p], zslice(j), sw[p])
            if k == 0:
                @pl.when(iv > 0)
                def _():
                    pltpu.make_async_copy(ar[q], zslice(j - 1), sw[q]).wait()
            else:
                pltpu.make_async_copy(ar[q], zslice(j - 1), sw[q]).wait()

            @pl.when(j < NCHUNKS - 3)
            def _():
                start_gather(j + 3, q)

    last = NCHUNKS - 1
    pltpu.make_async_copy(ar[3], zslice(last - 1), sw[3]).wait()
    wait_gather(last, 0)
    compute(0)
    pltpu.sync_copy(ar[0], zslice(last))


@functools.partial(
    pl.kernel,
    out_type=jax.ShapeDtypeStruct((NC, N_PAD, D), jnp.float32),
    mesh=_mesh,
    scratch_types=[
        pltpu.VMEM((SNCHUNKS, SCHUNK), jnp.int32),
        [pltpu.VMEM((SCHUNK, D), jnp.float32)] * 3,
        pltpu.VMEM_SHARED((N_PAD, D), jnp.float32),
        [pltpu.SemaphoreType.DMA] * 3,
        [pltpu.SemaphoreType.DMA] * 3,
    ],
)
def _edge_scatter(m_hbm, dst3, init_hbm, out_hbm, idxd, mr, acc, sm, ss):
    c = lax.axis_index("c")
    s = lax.axis_index("s")
    wid = s * NC + c
    base = wid * EPW

    pltpu.sync_copy(init_hbm.at[c, pl.ds(s * RPS, RPS)], acc.at[pl.ds(s * RPS, RPS)])
    pltpu.sync_copy(dst3.at[wid], idxd)
    plsc.subcore_barrier()

    def mslice(i):
        return m_hbm.at[pl.ds(base + i * SCHUNK, SCHUNK)]

    def start_add(j, p):
        pltpu.async_copy(mr[p], acc.at[idxd.at[j]], ss[p], add=True)

    def wait_add(j, p):
        pltpu.make_async_copy(mr[p], acc.at[idxd.at[j]], ss[p]).wait()

    pltpu.async_copy(mslice(0), mr[0], sm[0])
    pltpu.async_copy(mslice(1), mr[1], sm[1])

    @pl.loop(0, SNCHUNKS - 2, step=3)
    def ring_body(iv):
        for k in range(3):
            p = k % 3
            r = (k + 2) % 3
            j = iv + k
            pltpu.make_async_copy(mslice(j), mr[p], sm[p]).wait()
            start_add(j, p)
            if k == 0:
                @pl.when(iv > 0)
                def _():
                    wait_add(j - 1, r)
            else:
                wait_add(j - 1, r)
            pltpu.async_copy(mslice(j + 2), mr[r], sm[r])

    last = SNCHUNKS - 1
    pltpu.make_async_copy(mslice(last - 1), mr[0], sm[0]).wait()
    start_add(last - 1, 0)
    wait_add(last - 2, 2)
    pltpu.make_async_copy(mslice(last), mr[1], sm[1]).wait()
    start_add(last, 1)
    wait_add(last - 1, 0)
    wait_add(last, 1)

    plsc.subcore_barrier()
    pltpu.sync_copy(acc.at[pl.ds(s * RPS, RPS)], out_hbm.at[c, pl.ds(s * RPS, RPS)])


def _first_stage(x, w1a, w1b, b1r):
    br = 2000

    def body(x_ref, wa_ref, wb_ref, b_ref, a_ref, bo_ref):
        xv = x_ref[...]
        a_ref[...] = jnp.dot(xv, wa_ref[...], preferred_element_type=jnp.float32)
        bo_ref[...] = jnp.dot(xv, wb_ref[...], preferred_element_type=jnp.float32) + b_ref[...]

    return pl.pallas_call(
        body,
        grid=(N // br,),
        in_specs=[
            pl.BlockSpec((br, D), lambda i: (i, 0)),
            pl.BlockSpec((D, H), lambda i: (0, 0)),
            pl.BlockSpec((D, H), lambda i: (0, 0)),
            pl.BlockSpec((1, H), lambda i: (0, 0)),
        ],
        out_specs=[
            pl.BlockSpec((br, H), lambda i: (i, 0)),
            pl.BlockSpec((br, H), lambda i: (i, 0)),
        ],
        out_shape=[
            jax.ShapeDtypeStruct((N, H), jnp.float32),
            jax.ShapeDtypeStruct((N, H), jnp.float32),
        ],
    )(x, w1a, w1b, b1r)


def _message_mm(z, w2, b2r):
    br = 2000

    def body(z_ref, w_ref, b_ref, o_ref):
        o_ref[...] = jnp.dot(z_ref[...], w_ref[...], preferred_element_type=jnp.float32) + b_ref[...]

    return pl.pallas_call(
        body,
        grid=(ESEG // br,),
        in_specs=[
            pl.BlockSpec((br, H), lambda i: (i, 0)),
            pl.BlockSpec((H, D), lambda i: (0, 0)),
            pl.BlockSpec((1, D), lambda i: (0, 0)),
        ],
        out_specs=pl.BlockSpec((br, D), lambda i: (i, 0)),
        out_shape=jax.ShapeDtypeStruct((ESEG, D), jnp.float32),
    )(z, w2, b2r)


def _update_next(p0, p1, h, wu1a, wu1b, bu1r, wu2, bu2r, w1a, w1b, b1r):
    br = 2000

    def body(p0_ref, p1_ref, h_ref, wu1a_ref, wu1b_ref,
             bu1_ref, wu2_ref, bu2_ref, w1a_ref, w1b_ref, b1_ref,
             h_out, a_out, b_out):
        agg = p0_ref[...] + p1_ref[...]
        u = jnp.maximum(
            jnp.dot(agg, wu1a_ref[...], preferred_element_type=jnp.float32)
            + jnp.dot(h_ref[...], wu1b_ref[...], preferred_element_type=jnp.float32)
            + bu1_ref[...], 0.0)
        hn = jnp.dot(u, wu2_ref[...], preferred_element_type=jnp.float32) + bu2_ref[...]
        h_out[...] = hn
        a_out[...] = jnp.dot(hn, w1a_ref[...], preferred_element_type=jnp.float32)
        b_out[...] = jnp.dot(hn, w1b_ref[...], preferred_element_type=jnp.float32) + b1_ref[...]

    full = lambda i: (0, 0)
    return pl.pallas_call(
        body,
        grid=(N // br,),
        in_specs=[
            pl.BlockSpec((br, D), lambda i: (i, 0)),
            pl.BlockSpec((br, D), lambda i: (i, 0)),
            pl.BlockSpec((br, D), lambda i: (i, 0)),
            pl.BlockSpec((D, H), full),
            pl.BlockSpec((D, H), full),
            pl.BlockSpec((1, H), full),
            pl.BlockSpec((H, D), full),
            pl.BlockSpec((1, D), full),
            pl.BlockSpec((D, H), full),
            pl.BlockSpec((D, H), full),
            pl.BlockSpec((1, H), full),
        ],
        out_specs=[
            pl.BlockSpec((br, D), lambda i: (i, 0)),
            pl.BlockSpec((br, H), lambda i: (i, 0)),
            pl.BlockSpec((br, H), lambda i: (i, 0)),
        ],
        out_shape=[
            jax.ShapeDtypeStruct((N, D), jnp.float32),
            jax.ShapeDtypeStruct((N, H), jnp.float32),
            jax.ShapeDtypeStruct((N, H), jnp.float32),
        ],
    )(p0, p1, h, wu1a, wu1b, bu1r, wu2, bu2r, w1a, w1b, b1r)


def _update_readout(p0, p1, h, wu1a, wu1b, bu1r, wu2, bu2r, wr_pad, br_pad):
    br = 2000

    def body(p0_ref, p1_ref, h_ref, wu1a_ref, wu1b_ref,
             bu1_ref, wu2_ref, bu2_ref, wr_ref, brd_ref, o_ref):
        agg = p0_ref[...] + p1_ref[...]
        u = jnp.maximum(
            jnp.dot(agg, wu1a_ref[...], preferred_element_type=jnp.float32)
            + jnp.dot(h_ref[...], wu1b_ref[...], preferred_element_type=jnp.float32)
            + bu1_ref[...], 0.0)
        hn = jnp.dot(u, wu2_ref[...], preferred_element_type=jnp.float32) + bu2_ref[...]
        o_ref[...] = jnp.dot(hn, wr_ref[...], preferred_element_type=jnp.float32) + brd_ref[...]

    full = lambda i: (0, 0)
    return pl.pallas_call(
        body,
        grid=(N // br,),
        in_specs=[
            pl.BlockSpec((br, D), lambda i: (i, 0)),
            pl.BlockSpec((br, D), lambda i: (i, 0)),
            pl.BlockSpec((br, D), lambda i: (i, 0)),
            pl.BlockSpec((D, H), full),
            pl.BlockSpec((D, H), full),
            pl.BlockSpec((1, H), full),
            pl.BlockSpec((H, D), full),
            pl.BlockSpec((1, D), full),
            pl.BlockSpec((D, D), full),
            pl.BlockSpec((1, D), full),
        ],
        out_specs=pl.BlockSpec((br, D), lambda i: (i, 0)),
        out_shape=jax.ShapeDtypeStruct((N, D), jnp.float32),
    )(p0, p1, h, wu1a, wu1b, bu1r, wu2, bu2r, wr_pad, br_pad)


def kernel(x, edge_index, W1, b1, W2, b2, Wu1, bu1, Wu2, bu2, Wr, br):
    src4 = edge_index[0].reshape(S, NW, NCHUNKS, CHUNK)
    dst4 = edge_index[1].reshape(S, NW, NCHUNKS, CHUNK)
    dst4s = edge_index[1].reshape(S, NW, SNCHUNKS, SCHUNK)
    w1a, w1b = W1[:D], W1[D:]
    wu1a, wu1b = Wu1[:D], Wu1[D:]
    b1r = b1.reshape(1, H)
    b2r = b2.reshape(1, D)
    bu1r = bu1.reshape(1, H)
    bu2r = bu2.reshape(1, D)
    wr_pad = jnp.pad(Wr, ((0, 0), (0, D - 1)))
    br_pad = jnp.pad(br, (0, D - 1)).reshape(1, D)
    zero_nd = jnp.zeros((NC, N_PAD, D), jnp.float32)

    h = x
    a, b = _first_stage(x, w1a, w1b, b1r)
    for t in range(T):
        parts = zero_nd
        for sidx in range(S):
            z = _edge_gather(a, b, src4[sidx], dst4[sidx])
            m = _message_mm(z, W2, b2r)
            parts = _edge_scatter(m, dst4s[sidx], parts)
        p0, p1 = parts[0, :N], parts[1, :N]
        if t < T - 1:
            h, a, b = _update_next(p0, p1, h, wu1a, wu1b, bu1r, Wu2, bu2r,
                                   w1a, w1b, b1r)
        else:
            out_full = _update_readout(p0, p1, h, wu1a, wu1b, bu1r, Wu2, bu2r,
                                       wr_pad, br_pad)
    return out_full[:, :1]
